# baseline (speedup 1.0000x reference)
import functools

import jax
import jax.numpy as jnp
from jax import lax
from jax.experimental import pallas as pl
from jax.experimental.pallas import tpu as pltpu
from jax.experimental.pallas import tpu_sc as plsc

_NUM_ENTITY = 1000000
_DIM = 32
_BATCH = 16384
_EPR = 128 // _DIM

_info = plsc.get_sparse_core_info()
_NC = _info.num_cores
_NS = _info.num_subcores
_L = _info.num_lanes
_NW = _NC * _NS
_BPW = _BATCH // _NW
_HALF = _BPW // 2
_CHUNK = 128
_GROUPS = _HALF // _L

_mesh = plsc.VectorSubcoreMesh(core_axis_name="c", subcore_axis_name="s")


@functools.partial(
    pl.kernel,
    mesh=_mesh,
    compiler_params=pltpu.CompilerParams(
        needs_layout_passes=False, use_tc_tiling_on_sc=True),
    out_type=jax.ShapeDtypeStruct((_BATCH,), jnp.float32),
    scratch_types=[
        pltpu.VMEM((_BPW,), jnp.int32),
        pltpu.VMEM((_BPW,), jnp.int32),
        pltpu.VMEM((_HALF, 128), jnp.float32),
        pltpu.VMEM((_HALF, 128), jnp.float32),
        pltpu.VMEM((_HALF,), jnp.int32),
        pltpu.VMEM((_HALF,), jnp.int32),
        pltpu.VMEM((_BPW,), jnp.float32),
        pltpu.SemaphoreType.DMA,
        pltpu.SemaphoreType.DMA,
    ],
)
def _sc_pair_dot(idx0_hbm, idx1_hbm, win_hbm, wout_hbm, out_hbm,
                 idx0_v, idx1_v, in_bl, out_bl, row0_v, row1_v, res_v,
                 sem_a, sem_b):
    wid = lax.axis_index("s") * _NC + lax.axis_index("c")
    base = wid * _BPW

    pltpu.sync_copy(idx0_hbm.at[pl.ds(base, _BPW)], idx0_v)
    pltpu.sync_copy(idx1_hbm.at[pl.ds(base, _BPW)], idx1_v)

    lanes = lax.iota(jnp.int32, _L)

    for half in range(2):
        hoff = half * _HALF
        def rows_body(g, carry):
            sl = pl.ds(g * _L, _L)
            row0_v[sl] = lax.shift_right_logical(
                idx0_v[pl.ds(hoff + g * _L, _L)], 2)
            row1_v[sl] = lax.shift_right_logical(
                idx1_v[pl.ds(hoff + g * _L, _L)], 2)
            return carry

        lax.fori_loop(0, _GROUPS, rows_body, 0)

        copies = []
        for k in range(_HALF // _CHUNK):
            sl = pl.ds(k * _CHUNK, _CHUNK)
            copies.append(pltpu.async_copy(
                win_hbm.at[row0_v.at[sl]], in_bl.at[sl], sem_a))
            copies.append(pltpu.async_copy(
                wout_hbm.at[row1_v.at[sl]], out_bl.at[sl], sem_b))
        for cp in copies:
            cp.wait()

        def dot_body(g, carry):
            i_vec = g * _L + lanes
            a0 = lax.bitwise_and(idx0_v[pl.ds(hoff + g * _L, _L)], _EPR - 1)
            a1 = lax.bitwise_and(idx1_v[pl.ds(hoff + g * _L, _L)], _EPR - 1)
            col0 = a0 * _DIM
            col1 = a1 * _DIM
            acc = jnp.zeros((_L,), jnp.float32)
            for d in range(_DIM):
                va = plsc.load_gather(in_bl, [i_vec, col0 + d])
                vb = plsc.load_gather(out_bl, [i_vec, col1 + d])
                acc = acc + va * vb
            res_v[pl.ds(hoff + g * _L, _L)] = acc
            return carry

        lax.fori_loop(0, _GROUPS, dot_body, 0)

    pltpu.sync_copy(res_v, out_hbm.at[pl.ds(base, _BPW)])


def kernel(idxs, W_in, W_out):
    idx0 = idxs[:, 0].astype(jnp.int32)
    idx1 = idxs[:, 1].astype(jnp.int32)
    w_in_r = W_in.reshape(_NUM_ENTITY // _EPR, 128)
    w_out_r = W_out.reshape(_NUM_ENTITY // _EPR, 128)
    return _sc_pair_dot(idx0, idx1, w_in_r, w_out_r)
